# baseline (speedup 1.0000x reference)
import functools

import jax
import jax.numpy as jnp
import numpy as np
from jax import lax
from jax.experimental import pallas as pl
from jax.experimental.pallas import tpu as pltpu
from jax.experimental.pallas import tpu_sc as plsc

N = 10000
E = 320000
D = 128

N_PAD = 10240
PAD_ROWS = N_PAD - N
NC = 2
NS = 16
NW = NC * NS
CH = 80
CPW = 128
NBUF = 4
NSTG = 4
NH = CPW // NSTG
EPW = CPW * CH
ROWS_SUB = N_PAD // NS

_MESH = plsc.VectorSubcoreMesh(core_axis_name="c", subcore_axis_name="s")


def _zero_vmem_f32(ref2d, rows, cols):
    zeros16 = jnp.zeros((16,), jnp.float32)

    def body(i, _):
        for k in range(cols // 16):
            ref2d[i, pl.ds(k * 16, 16)] = zeros16
        return 0

    lax.fori_loop(0, rows, body, 0)


@functools.partial(
    pl.kernel,
    out_type=jax.ShapeDtypeStruct((2, N_PAD), jnp.float32),
    mesh=_MESH,
    scratch_types=[
        pltpu.VMEM_SHARED((N_PAD,), jnp.float32),
        pltpu.VMEM((EPW,), jnp.int32),
        pltpu.VMEM((CH,), jnp.float32),
        pltpu.VMEM((ROWS_SUB,), jnp.float32),
    ] + [pltpu.SemaphoreType.DMA] * 4,
)
def _sc_degree(ei_hbm, out_hbm, acc, dst_v, ones_v, stage_v, *dsems):
    c = lax.axis_index("c")
    s = lax.axis_index("s")
    w = s * NC + c

    ones16 = jnp.ones((16,), jnp.float32)
    zeros16 = jnp.zeros((16,), jnp.float32)
    for k in range(CH // 16):
        ones_v[pl.ds(k * 16, 16)] = ones16

    def zbody(i, _):
        stage_v[pl.ds(i * 16, 16)] = zeros16
        return 0

    lax.fori_loop(0, ROWS_SUB // 16, zbody, 0)
    pltpu.sync_copy(stage_v, acc.at[pl.ds(s * ROWS_SUB, ROWS_SUB)])
    pltpu.sync_copy(ei_hbm.at[1].at[w], dst_v)
    plsc.subcore_barrier()

    def body(t, _):
        for b in range(4):
            idx = dst_v.at[pl.ds((t * 4 + b) * CH, CH)]
            pltpu.async_copy(ones_v, acc.at[idx], dsems[b], add=True)
        for b in range(4):
            pltpu.make_async_copy(ones_v, acc.at[dst_v.at[pl.ds(0, CH)]], dsems[b]).wait()
        return 0

    lax.fori_loop(0, CPW // 4, body, 0)
    plsc.subcore_barrier()

    sl = pl.ds(s * ROWS_SUB, ROWS_SUB)
    pltpu.sync_copy(acc.at[sl], stage_v)
    pltpu.sync_copy(stage_v, out_hbm.at[c].at[sl])


@functools.partial(
    pl.kernel,
    out_type=jax.ShapeDtypeStruct((2, N_PAD, D), jnp.float32),
    mesh=_MESH,
    scratch_types=[
        pltpu.VMEM_SHARED((N_PAD, D), jnp.float32),
        pltpu.VMEM((NH * CH,), jnp.int32),
        pltpu.VMEM((NH * CH,), jnp.int32),
        pltpu.VMEM((NBUF, CH, D), jnp.float32),
    ] + [pltpu.SemaphoreType.DMA] * (2 * NBUF),
)
def _sc_agg(table_hbm, ei_hbm, out_hbm,
            acc, src_v, dst_v, rows, *sems):
    c = lax.axis_index("c")
    s = lax.axis_index("s")
    w = s * NC + c
    gsems = sems[:NBUF]
    ssems = sems[NBUF:]

    _zero_vmem_f32(rows.at[0], CH, D)
    for t in range(ROWS_SUB // CH):
        pltpu.sync_copy(rows.at[0], acc.at[pl.ds(s * ROWS_SUB + t * CH, CH)])
    plsc.subcore_barrier()

    def wait_gather(b):
        pltpu.make_async_copy(
            table_hbm.at[src_v.at[pl.ds(0, CH)]], rows.at[b], gsems[b]).wait()

    def wait_scatter(b):
        pltpu.make_async_copy(
            rows.at[b], acc.at[dst_v.at[pl.ds(0, CH)]], ssems[b]).wait()

    def body(t, _):
        for b in range(NBUF):
            jj = t * NBUF + b
            wait_gather(b)
            idx = dst_v.at[pl.ds(jj * CH, CH)]
            pltpu.async_copy(rows.at[b], acc.at[idx], ssems[b], add=True)
        for b in range(NBUF):
            jj = t * NBUF + b + NBUF

            @pl.when(jj < NH)
            def _():
                wait_scatter(b)
                idx = src_v.at[pl.ds(jj * CH, CH)]
                pltpu.async_copy(table_hbm.at[idx], rows.at[b], gsems[b])

        return 0

    for stage in range(NSTG):
        pltpu.sync_copy(ei_hbm.at[0].at[w].at[pl.ds(stage * NH * CH, NH * CH)], src_v)
        pltpu.sync_copy(ei_hbm.at[1].at[w].at[pl.ds(stage * NH * CH, NH * CH)], dst_v)
        for b in range(NBUF):
            pltpu.async_copy(
                table_hbm.at[src_v.at[pl.ds(b * CH, CH)]], rows.at[b], gsems[b])
        lax.fori_loop(0, NH // NBUF, body, 0)
        for b in range(NBUF):
            wait_scatter(b)
    plsc.subcore_barrier()

    for t in range(ROWS_SUB // CH):
        sl = pl.ds(s * ROWS_SUB + t * CH, CH)
        pltpu.sync_copy(acc.at[sl], out_hbm.at[c].at[sl])


BR = 1024
BR_C = 1000


def _dinv_block(dga_ref, dgb_ref, row0, masked):
    deg = dga_ref[0] + dgb_ref[0] + 1.0
    dinv = lax.rsqrt(deg)
    if masked:
        rows = lax.broadcasted_iota(jnp.int32, deg.shape, 0) + row0
        dinv = jnp.where(rows < N, dinv, 0.0)
    return dinv


def _mm(a, b):
    return lax.dot_general(a, b, (((1,), (0,)), ((), ())),
                           precision=lax.Precision.HIGHEST,
                           preferred_element_type=jnp.float32)


def _tc_a_body(x_ref, w_ref, dga_ref, dgb_ref, o_ref):
    deg = dga_ref[0] + dgb_ref[0] + 1.0
    rows = lax.broadcasted_iota(jnp.int32, deg.shape, 0) + pl.program_id(0) * BR
    h = _mm(x_ref[...], w_ref[...]) * lax.rsqrt(deg)
    o_ref[...] = jnp.where(rows < N, h, 0.0)


def _tc_b_body(h_ref, aga_ref, agb_ref, dga_ref, dgb_ref, b_ref, w_ref, o_ref):
    dinv = _dinv_block(dga_ref, dgb_ref, pl.program_id(0) * BR, True)
    z = dinv * (aga_ref[0] + agb_ref[0] + h_ref[...]) + b_ref[...]
    z = jnp.maximum(z, 0.0)
    o_ref[...] = _mm(z, w_ref[...]) * dinv


def _tc_c_body(h_ref, aga_ref, agb_ref, dga_ref, dgb_ref, b_ref, o_ref):
    dinv = _dinv_block(dga_ref, dgb_ref, 0, False)
    o_ref[...] = dinv * (aga_ref[0] + agb_ref[0] + h_ref[...]) + b_ref[...]


def _row_spec(br):
    return pl.BlockSpec((br, D), lambda i: (i, 0))


def _deg_specs(br):
    return [pl.BlockSpec((1, br, 1), lambda i: (0, i, 0)),
            pl.BlockSpec((1, br, 1), lambda i: (1, i, 0))]


def _agg_specs(br):
    return [pl.BlockSpec((1, br, D), lambda i: (0, i, 0)),
            pl.BlockSpec((1, br, D), lambda i: (1, i, 0))]


_W_SPEC = pl.BlockSpec((D, D), lambda i: (0, 0))
_B_SPEC = pl.BlockSpec((1, D), lambda i: (0, 0))


def _tc_a(x_pad, w1, deg):
    return pl.pallas_call(
        _tc_a_body,
        grid=(N_PAD // BR,),
        in_specs=[_row_spec(BR), _W_SPEC] + _deg_specs(BR),
        out_specs=_row_spec(BR),
        out_shape=jax.ShapeDtypeStruct((N_PAD, D), jnp.float32),
    )(x_pad, w1, deg, deg)


def _tc_b(h1s, agg, deg, b1, w2):
    return pl.pallas_call(
        _tc_b_body,
        grid=(N_PAD // BR,),
        in_specs=([_row_spec(BR)] + _agg_specs(BR) + _deg_specs(BR)
                  + [_B_SPEC, _W_SPEC]),
        out_specs=_row_spec(BR),
        out_shape=jax.ShapeDtypeStruct((N_PAD, D), jnp.float32),
    )(h1s, agg, agg, deg, deg, b1, w2)


def _tc_c(h2s, agg, deg, b2):
    return pl.pallas_call(
        _tc_c_body,
        grid=(N // BR_C,),
        in_specs=[_row_spec(BR_C)] + _agg_specs(BR_C) + _deg_specs(BR_C) + [_B_SPEC],
        out_specs=_row_spec(BR_C),
        out_shape=jax.ShapeDtypeStruct((N, D), jnp.float32),
    )(h2s, agg, agg, deg, deg, b2)


_PAD_IDX = np.broadcast_to(
    np.asarray(N + np.arange(PAD_ROWS), dtype=np.int32), (2, NW, PAD_ROWS))


def kernel(x, edge_index, W1, b1, W2, b2):
    ei = jnp.concatenate(
        [edge_index.reshape(2, NW, E // NW), jnp.asarray(_PAD_IDX)], axis=2)
    b1r = b1.reshape(1, D)
    b2r = b2.reshape(1, D)

    deg = _sc_degree(ei).reshape(2, N_PAD, 1)
    h1s = _tc_a(x, W1, deg)
    agg1 = _sc_agg(h1s, ei)
    h2s = _tc_b(h1s, agg1, deg, b1r, W2)
    agg2 = _sc_agg(h2s, ei)
    return _tc_c(h2s, agg2, deg, b2r)


# speedup vs baseline: 1.0200x; 1.0200x over previous
import functools

import jax
import jax.numpy as jnp
import numpy as np
from jax import lax
from jax.experimental import pallas as pl
from jax.experimental.pallas import tpu as pltpu
from jax.experimental.pallas import tpu_sc as plsc

N = 10000
E = 320000
D = 128

N_PAD = 10240
PAD_ROWS = N_PAD - N
NC = 2
NS = 16
NW = NC * NS
CH = 80
CPW = 128
NBUF = 4
NSTG = 4
NH = CPW // NSTG
EPW = CPW * CH
ROWS_SUB = N_PAD // NS

_MESH = plsc.VectorSubcoreMesh(core_axis_name="c", subcore_axis_name="s")


def _zero_vmem_f32(ref2d, rows, cols):
    zeros16 = jnp.zeros((16,), jnp.float32)

    def body(i, _):
        for k in range(cols // 16):
            ref2d[i, pl.ds(k * 16, 16)] = zeros16
        return 0

    lax.fori_loop(0, rows, body, 0)


@functools.partial(
    pl.kernel,
    out_type=jax.ShapeDtypeStruct((2, N_PAD), jnp.float32),
    mesh=_MESH,
    scratch_types=[
        pltpu.VMEM_SHARED((N_PAD,), jnp.float32),
        pltpu.VMEM((EPW,), jnp.int32),
        pltpu.VMEM((CH,), jnp.float32),
        pltpu.VMEM((ROWS_SUB,), jnp.float32),
    ] + [pltpu.SemaphoreType.DMA] * 4,
)
def _sc_degree(ei_hbm, out_hbm, acc, dst_v, ones_v, stage_v, *dsems):
    c = lax.axis_index("c")
    s = lax.axis_index("s")
    w = s * NC + c

    ones16 = jnp.ones((16,), jnp.float32)
    zeros16 = jnp.zeros((16,), jnp.float32)
    for k in range(CH // 16):
        ones_v[pl.ds(k * 16, 16)] = ones16

    def zbody(i, _):
        stage_v[pl.ds(i * 16, 16)] = zeros16
        return 0

    lax.fori_loop(0, ROWS_SUB // 16, zbody, 0)
    pltpu.sync_copy(stage_v, acc.at[pl.ds(s * ROWS_SUB, ROWS_SUB)])
    pltpu.sync_copy(ei_hbm.at[1].at[w], dst_v)
    plsc.subcore_barrier()

    def body(t, _):
        for b in range(4):
            idx = dst_v.at[pl.ds((t * 4 + b) * CH, CH)]
            pltpu.async_copy(ones_v, acc.at[idx], dsems[b], add=True)
        for b in range(4):
            pltpu.make_async_copy(ones_v, acc.at[dst_v.at[pl.ds(0, CH)]], dsems[b]).wait()
        return 0

    lax.fori_loop(0, CPW // 4, body, 0)
    plsc.subcore_barrier()

    sl = pl.ds(s * ROWS_SUB, ROWS_SUB)
    pltpu.sync_copy(acc.at[sl], stage_v)
    pltpu.sync_copy(stage_v, out_hbm.at[c].at[sl])


@functools.partial(
    pl.kernel,
    out_type=jax.ShapeDtypeStruct((2, N_PAD, D), jnp.float32),
    mesh=_MESH,
    scratch_types=[
        pltpu.VMEM_SHARED((N_PAD, D), jnp.float32),
        pltpu.VMEM((NH * CH,), jnp.int32),
        pltpu.VMEM((NH * CH,), jnp.int32),
        pltpu.VMEM((NBUF, CH, D), jnp.float32),
    ] + [pltpu.SemaphoreType.DMA] * (2 * NBUF),
)
def _sc_agg(table_hbm, ei_hbm, out_hbm,
            acc, src_v, dst_v, rows, *sems):
    c = lax.axis_index("c")
    s = lax.axis_index("s")
    w = s * NC + c
    gsems = sems[:NBUF]
    ssems = sems[NBUF:]

    _zero_vmem_f32(rows.at[0], CH, D)
    for t in range(ROWS_SUB // CH):
        pltpu.sync_copy(rows.at[0], acc.at[pl.ds(s * ROWS_SUB + t * CH, CH)])
    plsc.subcore_barrier()

    def wait_gather(b):
        pltpu.make_async_copy(
            table_hbm.at[src_v.at[pl.ds(0, CH)]], rows.at[b], gsems[b]).wait()

    def wait_scatter(b):
        pltpu.make_async_copy(
            rows.at[b], acc.at[dst_v.at[pl.ds(0, CH)]], ssems[b]).wait()

    def body(t, _):
        for b in range(NBUF):
            jj = t * NBUF + b
            wait_gather(b)
            idx = dst_v.at[pl.ds(jj * CH, CH)]
            pltpu.async_copy(rows.at[b], acc.at[idx], ssems[b], add=True)
        for b in range(NBUF):
            jj = t * NBUF + b + NBUF

            @pl.when(jj < NH)
            def _():
                wait_scatter(b)
                idx = src_v.at[pl.ds(jj * CH, CH)]
                pltpu.async_copy(table_hbm.at[idx], rows.at[b], gsems[b])

        return 0

    for stage in range(NSTG):
        pltpu.sync_copy(ei_hbm.at[0].at[w].at[pl.ds(stage * NH * CH, NH * CH)], src_v)
        pltpu.sync_copy(ei_hbm.at[1].at[w].at[pl.ds(stage * NH * CH, NH * CH)], dst_v)
        for b in range(NBUF):
            pltpu.async_copy(
                table_hbm.at[src_v.at[pl.ds(b * CH, CH)]], rows.at[b], gsems[b])
        lax.fori_loop(0, NH // NBUF, body, 0)
        for b in range(NBUF):
            wait_scatter(b)
    plsc.subcore_barrier()

    for t in range(ROWS_SUB // CH):
        sl = pl.ds(s * ROWS_SUB + t * CH, CH)
        pltpu.sync_copy(acc.at[sl], out_hbm.at[c].at[sl])


BR = 2048
BR_C = 2000


def _dinv_block(dga_ref, dgb_ref, row0, masked):
    deg = dga_ref[0] + dgb_ref[0] + 1.0
    dinv = lax.rsqrt(deg)
    if masked:
        rows = lax.broadcasted_iota(jnp.int32, deg.shape, 0) + row0
        dinv = jnp.where(rows < N, dinv, 0.0)
    return dinv


def _mm(a, b):
    return lax.dot_general(a, b, (((1,), (0,)), ((), ())),
                           precision=lax.Precision.HIGHEST,
                           preferred_element_type=jnp.float32)


def _tc_a_body(x_ref, w_ref, dga_ref, dgb_ref, o_ref):
    deg = dga_ref[0] + dgb_ref[0] + 1.0
    rows = lax.broadcasted_iota(jnp.int32, deg.shape, 0) + pl.program_id(0) * BR
    h = _mm(x_ref[...], w_ref[...]) * lax.rsqrt(deg)
    o_ref[...] = jnp.where(rows < N, h, 0.0)


def _tc_b_body(h_ref, aga_ref, agb_ref, dga_ref, dgb_ref, b_ref, w_ref, o_ref):
    dinv = _dinv_block(dga_ref, dgb_ref, pl.program_id(0) * BR, True)
    z = dinv * (aga_ref[0] + agb_ref[0] + h_ref[...]) + b_ref[...]
    z = jnp.maximum(z, 0.0)
    o_ref[...] = _mm(z, w_ref[...]) * dinv


def _tc_c_body(h_ref, aga_ref, agb_ref, dga_ref, dgb_ref, b_ref, o_ref):
    dinv = _dinv_block(dga_ref, dgb_ref, 0, False)
    o_ref[...] = dinv * (aga_ref[0] + agb_ref[0] + h_ref[...]) + b_ref[...]


def _row_spec(br):
    return pl.BlockSpec((br, D), lambda i: (i, 0))


def _deg_specs(br):
    return [pl.BlockSpec((1, br, 1), lambda i: (0, i, 0)),
            pl.BlockSpec((1, br, 1), lambda i: (1, i, 0))]


def _agg_specs(br):
    return [pl.BlockSpec((1, br, D), lambda i: (0, i, 0)),
            pl.BlockSpec((1, br, D), lambda i: (1, i, 0))]


_W_SPEC = pl.BlockSpec((D, D), lambda i: (0, 0))
_B_SPEC = pl.BlockSpec((1, D), lambda i: (0, 0))


def _tc_a(x_pad, w1, deg):
    return pl.pallas_call(
        _tc_a_body,
        grid=(N_PAD // BR,),
        in_specs=[_row_spec(BR), _W_SPEC] + _deg_specs(BR),
        out_specs=_row_spec(BR),
        out_shape=jax.ShapeDtypeStruct((N_PAD, D), jnp.float32),
    )(x_pad, w1, deg, deg)


def _tc_b(h1s, agg, deg, b1, w2):
    return pl.pallas_call(
        _tc_b_body,
        grid=(N_PAD // BR,),
        in_specs=([_row_spec(BR)] + _agg_specs(BR) + _deg_specs(BR)
                  + [_B_SPEC, _W_SPEC]),
        out_specs=_row_spec(BR),
        out_shape=jax.ShapeDtypeStruct((N_PAD, D), jnp.float32),
    )(h1s, agg, agg, deg, deg, b1, w2)


def _tc_c(h2s, agg, deg, b2):
    return pl.pallas_call(
        _tc_c_body,
        grid=(N // BR_C,),
        in_specs=[_row_spec(BR_C)] + _agg_specs(BR_C) + _deg_specs(BR_C) + [_B_SPEC],
        out_specs=_row_spec(BR_C),
        out_shape=jax.ShapeDtypeStruct((N, D), jnp.float32),
    )(h2s, agg, agg, deg, deg, b2)


_PAD_IDX = np.broadcast_to(
    np.asarray(N + np.arange(PAD_ROWS), dtype=np.int32), (2, NW, PAD_ROWS))


def kernel(x, edge_index, W1, b1, W2, b2):
    ei = jnp.concatenate(
        [edge_index.reshape(2, NW, E // NW), jnp.asarray(_PAD_IDX)], axis=2)
    b1r = b1.reshape(1, D)
    b2r = b2.reshape(1, D)

    deg = _sc_degree(ei).reshape(2, N_PAD, 1)
    h1s = _tc_a(x, W1, deg)
    agg1 = _sc_agg(h1s, ei)
    h2s = _tc_b(h1s, agg1, deg, b1r, W2)
    agg2 = _sc_agg(h2s, ei)
    return _tc_c(h2s, agg2, deg, b2r)


# speedup vs baseline: 1.0308x; 1.0105x over previous
import functools

import jax
import jax.numpy as jnp
import numpy as np
from jax import lax
from jax.experimental import pallas as pl
from jax.experimental.pallas import tpu as pltpu
from jax.experimental.pallas import tpu_sc as plsc

N = 10000
E = 320000
D = 128

N_PAD = 10240
PAD_ROWS = N_PAD - N
NC = 2
NS = 16
NW = NC * NS
CH = 80
CPW = 128
NBUF = 4
NSTG = 4
NH = CPW // NSTG
EPW = CPW * CH
ROWS_SUB = N_PAD // NS

_MESH = plsc.VectorSubcoreMesh(core_axis_name="c", subcore_axis_name="s")


def _zero_vmem_f32(ref2d, rows, cols):
    zeros16 = jnp.zeros((16,), jnp.float32)

    def body(i, _):
        for k in range(cols // 16):
            ref2d[i, pl.ds(k * 16, 16)] = zeros16
        return 0

    lax.fori_loop(0, rows, body, 0)


@functools.partial(
    pl.kernel,
    out_type=jax.ShapeDtypeStruct((2, N_PAD), jnp.float32),
    mesh=_MESH,
    scratch_types=[
        pltpu.VMEM_SHARED((N_PAD,), jnp.float32),
        pltpu.VMEM((EPW,), jnp.int32),
        pltpu.VMEM((CH,), jnp.float32),
        pltpu.VMEM((ROWS_SUB,), jnp.float32),
    ] + [pltpu.SemaphoreType.DMA] * 4,
)
def _sc_degree(ei_hbm, out_hbm, acc, dst_v, ones_v, stage_v, *dsems):
    c = lax.axis_index("c")
    s = lax.axis_index("s")
    w = s * NC + c

    ones16 = jnp.ones((16,), jnp.float32)
    zeros16 = jnp.zeros((16,), jnp.float32)
    for k in range(CH // 16):
        ones_v[pl.ds(k * 16, 16)] = ones16

    def zbody(i, _):
        stage_v[pl.ds(i * 16, 16)] = zeros16
        return 0

    lax.fori_loop(0, ROWS_SUB // 16, zbody, 0)
    pltpu.sync_copy(stage_v, acc.at[pl.ds(s * ROWS_SUB, ROWS_SUB)])
    pltpu.sync_copy(ei_hbm.at[1].at[w], dst_v)
    plsc.subcore_barrier()

    def body(t, _):
        for b in range(4):
            idx = dst_v.at[pl.ds((t * 4 + b) * CH, CH)]
            pltpu.async_copy(ones_v, acc.at[idx], dsems[b], add=True)
        for b in range(4):
            pltpu.make_async_copy(ones_v, acc.at[dst_v.at[pl.ds(0, CH)]], dsems[b]).wait()
        return 0

    lax.fori_loop(0, CPW // 4, body, 0)
    plsc.subcore_barrier()

    sl = pl.ds(s * ROWS_SUB, ROWS_SUB)
    pltpu.sync_copy(acc.at[sl], stage_v)
    pltpu.sync_copy(stage_v, out_hbm.at[c].at[sl])


@functools.partial(
    pl.kernel,
    out_type=jax.ShapeDtypeStruct((2, N_PAD, D), jnp.float32),
    mesh=_MESH,
    scratch_types=[
        pltpu.VMEM_SHARED((N_PAD, D), jnp.float32),
        pltpu.VMEM((NH * CH,), jnp.int32),
        pltpu.VMEM((NH * CH,), jnp.int32),
        pltpu.VMEM((NBUF, CH, D), jnp.float32),
    ] + [pltpu.SemaphoreType.DMA] * (2 * NBUF),
)
def _sc_agg(table_hbm, ei_hbm, out_hbm,
            acc, src_v, dst_v, rows, *sems):
    c = lax.axis_index("c")
    s = lax.axis_index("s")
    w = s * NC + c
    gsems = sems[:NBUF]
    ssems = sems[NBUF:]

    _zero_vmem_f32(rows.at[0], CH, D)
    for t in range(ROWS_SUB // CH):
        pltpu.sync_copy(rows.at[0], acc.at[pl.ds(s * ROWS_SUB + t * CH, CH)])
    plsc.subcore_barrier()

    def wait_gather(b):
        pltpu.make_async_copy(
            table_hbm.at[src_v.at[pl.ds(0, CH)]], rows.at[b], gsems[b]).wait()

    def wait_scatter(b):
        pltpu.make_async_copy(
            rows.at[b], acc.at[dst_v.at[pl.ds(0, CH)]], ssems[b]).wait()

    def body(t, _):
        for b in range(NBUF):
            jj = t * NBUF + b
            wait_gather(b)
            idx = dst_v.at[pl.ds(jj * CH, CH)]
            pltpu.async_copy(rows.at[b], acc.at[idx], ssems[b], add=True)
        for b in range(NBUF):
            jj = t * NBUF + b + NBUF

            @pl.when(jj < NH)
            def _():
                wait_scatter(b)
                idx = src_v.at[pl.ds(jj * CH, CH)]
                pltpu.async_copy(table_hbm.at[idx], rows.at[b], gsems[b])

        return 0

    for stage in range(NSTG):
        pltpu.sync_copy(ei_hbm.at[0].at[w].at[pl.ds(stage * NH * CH, NH * CH)], src_v)
        pltpu.sync_copy(ei_hbm.at[1].at[w].at[pl.ds(stage * NH * CH, NH * CH)], dst_v)
        for b in range(NBUF):
            pltpu.async_copy(
                table_hbm.at[src_v.at[pl.ds(b * CH, CH)]], rows.at[b], gsems[b])
        lax.fori_loop(0, NH // NBUF, body, 0)
        for b in range(NBUF):
            wait_scatter(b)
    plsc.subcore_barrier()

    for t in range(ROWS_SUB // CH):
        sl = pl.ds(s * ROWS_SUB + t * CH, CH)
        pltpu.sync_copy(acc.at[sl], out_hbm.at[c].at[sl])


BR = 2048
BR_C = 2000


def _dinv_block(dga_ref, dgb_ref, row0, masked):
    deg = dga_ref[0] + dgb_ref[0] + 1.0
    dinv = lax.rsqrt(deg)
    if masked:
        rows = lax.broadcasted_iota(jnp.int32, deg.shape, 0) + row0
        dinv = jnp.where(rows < N, dinv, 0.0)
    return dinv


def _mm(a, b):
    return lax.dot_general(a, b, (((1,), (0,)), ((), ())),
                           preferred_element_type=jnp.float32)


def _tc_a_body(x_ref, w_ref, dga_ref, dgb_ref, o_ref):
    deg = dga_ref[0] + dgb_ref[0] + 1.0
    rows = lax.broadcasted_iota(jnp.int32, deg.shape, 0) + pl.program_id(0) * BR
    h = _mm(x_ref[...], w_ref[...]) * lax.rsqrt(deg)
    o_ref[...] = jnp.where(rows < N, h, 0.0)


def _tc_b_body(h_ref, aga_ref, agb_ref, dga_ref, dgb_ref, b_ref, w_ref, o_ref):
    dinv = _dinv_block(dga_ref, dgb_ref, pl.program_id(0) * BR, True)
    z = dinv * (aga_ref[0] + agb_ref[0] + h_ref[...]) + b_ref[...]
    z = jnp.maximum(z, 0.0)
    o_ref[...] = _mm(z, w_ref[...]) * dinv


def _tc_c_body(h_ref, aga_ref, agb_ref, dga_ref, dgb_ref, b_ref, o_ref):
    dinv = _dinv_block(dga_ref, dgb_ref, 0, False)
    o_ref[...] = dinv * (aga_ref[0] + agb_ref[0] + h_ref[...]) + b_ref[...]


def _row_spec(br):
    return pl.BlockSpec((br, D), lambda i: (i, 0))


def _deg_specs(br):
    return [pl.BlockSpec((1, br, 1), lambda i: (0, i, 0)),
            pl.BlockSpec((1, br, 1), lambda i: (1, i, 0))]


def _agg_specs(br):
    return [pl.BlockSpec((1, br, D), lambda i: (0, i, 0)),
            pl.BlockSpec((1, br, D), lambda i: (1, i, 0))]


_W_SPEC = pl.BlockSpec((D, D), lambda i: (0, 0))
_B_SPEC = pl.BlockSpec((1, D), lambda i: (0, 0))


def _tc_a(x_pad, w1, deg):
    return pl.pallas_call(
        _tc_a_body,
        grid=(N_PAD // BR,),
        in_specs=[_row_spec(BR), _W_SPEC] + _deg_specs(BR),
        out_specs=_row_spec(BR),
        out_shape=jax.ShapeDtypeStruct((N_PAD, D), jnp.float32),
    )(x_pad, w1, deg, deg)


def _tc_b(h1s, agg, deg, b1, w2):
    return pl.pallas_call(
        _tc_b_body,
        grid=(N_PAD // BR,),
        in_specs=([_row_spec(BR)] + _agg_specs(BR) + _deg_specs(BR)
                  + [_B_SPEC, _W_SPEC]),
        out_specs=_row_spec(BR),
        out_shape=jax.ShapeDtypeStruct((N_PAD, D), jnp.float32),
    )(h1s, agg, agg, deg, deg, b1, w2)


def _tc_c(h2s, agg, deg, b2):
    return pl.pallas_call(
        _tc_c_body,
        grid=(N // BR_C,),
        in_specs=[_row_spec(BR_C)] + _agg_specs(BR_C) + _deg_specs(BR_C) + [_B_SPEC],
        out_specs=_row_spec(BR_C),
        out_shape=jax.ShapeDtypeStruct((N, D), jnp.float32),
    )(h2s, agg, agg, deg, deg, b2)


_PAD_IDX = np.broadcast_to(
    np.asarray(N + np.arange(PAD_ROWS), dtype=np.int32), (2, NW, PAD_ROWS))


def kernel(x, edge_index, W1, b1, W2, b2):
    ei = jnp.concatenate(
        [edge_index.reshape(2, NW, E // NW), jnp.asarray(_PAD_IDX)], axis=2)
    b1r = b1.reshape(1, D)
    b2r = b2.reshape(1, D)

    deg = _sc_degree(ei).reshape(2, N_PAD, 1)
    h1s = _tc_a(x, W1, deg)
    agg1 = _sc_agg(h1s, ei)
    h2s = _tc_b(h1s, agg1, deg, b1r, W2)
    agg2 = _sc_agg(h2s, ei)
    return _tc_c(h2s, agg2, deg, b2r)
